# Initial kernel scaffold; baseline (speedup 1.0000x reference)
import jax
import jax.numpy as jnp
from jax.experimental import pallas as pl


def kernel(x, edge_index, batch, W_in, b_in, W_c1, b_c1, W_c2, b_c2, W_out, b_out):
    raise NotImplementedError("write your pallas kernel here")


# speedup vs baseline: 2.3588x; 2.3588x over previous
import functools

import jax
import jax.numpy as jnp
from jax import lax
from jax.experimental import pallas as pl
from jax.experimental.pallas import tpu as pltpu
from jax.experimental.pallas import tpu_sc as plsc

N = 10000
NP = 10240
E = 320000
EP = 327680
G = 128
NCHUNK = EP // G
NWORK = 32
J = 4
GROUPS = NCHUNK // (NWORK * J)
IN_DIM = 128
HID = 64
N_ITERS = 10
N_GRAPHS = 16
OUT_DIM = 16

BE = 2048
BN = 1024

_mesh = plsc.VectorSubcoreMesh(core_axis_name="c", subcore_axis_name="s")
_sc_params = pltpu.CompilerParams(use_tc_tiling_on_sc=False)


@functools.partial(
    pl.kernel,
    out_type=[
        jax.ShapeDtypeStruct((EP, HID), jnp.float32),
        jax.ShapeDtypeStruct((EP, HID), jnp.float32),
    ],
    mesh=_mesh,
    compiler_params=_sc_params,
    scratch_types=[
        pltpu.VMEM((J, G), jnp.int32),
        pltpu.VMEM((J, G), jnp.int32),
        pltpu.VMEM((J * G, HID), jnp.float32),
        pltpu.VMEM((J * G, HID), jnp.float32),
        pltpu.SemaphoreType.DMA,
        pltpu.SemaphoreType.DMA,
    ],
)
def _sc_gather(p_hbm, q_hbm, dst_hbm, src_hbm, pg_hbm, qg_hbm,
               dv, sv, pr, qr, sem_p, sem_q):
    c = lax.axis_index("c")
    s = lax.axis_index("s")
    wid = s * 2 + c

    def body(g, carry):
        row0 = (wid * GROUPS + g) * J
        base = row0 * G
        pltpu.sync_copy(dst_hbm.at[pl.ds(row0, J)], dv)
        pltpu.sync_copy(src_hbm.at[pl.ds(row0, J)], sv)
        cps = []
        for j in range(J):
            cps.append(pltpu.async_copy(
                p_hbm.at[dv.at[j]], pr.at[pl.ds(j * G, G)], sem_p))
            cps.append(pltpu.async_copy(
                q_hbm.at[sv.at[j]], qr.at[pl.ds(j * G, G)], sem_q))
        for cp in cps:
            cp.wait()
        pltpu.sync_copy(pr, pg_hbm.at[pl.ds(base, J * G)])
        pltpu.sync_copy(qr, qg_hbm.at[pl.ds(base, J * G)])
        return carry

    lax.fori_loop(0, GROUPS, body, 0)


@functools.partial(
    pl.kernel,
    out_type=jax.ShapeDtypeStruct((2, NP, HID), jnp.float32),
    mesh=_mesh,
    compiler_params=_sc_params,
    scratch_types=[
        pltpu.VMEM((J, G), jnp.int32),
        pltpu.VMEM((J * G, HID), jnp.float32),
        pltpu.VMEM_SHARED((NP, HID), jnp.float32),
    ],
)
def _sc_scatter(u_hbm, dst_hbm, zeros_hbm, out_hbm, dv, uv, acc):
    c = lax.axis_index("c")
    s = lax.axis_index("s")
    rpt = NP // 16

    pltpu.sync_copy(zeros_hbm.at[pl.ds(s * rpt, rpt)],
                    acc.at[pl.ds(s * rpt, rpt)])
    plsc.subcore_barrier()

    def body(g, carry):
        row0 = ((c * 16 + s) * GROUPS + g) * J
        base = row0 * G
        pltpu.sync_copy(dst_hbm.at[pl.ds(row0, J)], dv)
        pltpu.sync_copy(u_hbm.at[pl.ds(base, J * G)], uv)
        for j in range(J):
            pltpu.sync_copy(uv.at[pl.ds(j * G, G)], acc.at[dv.at[j]],
                            add=True)
        return carry

    lax.fori_loop(0, GROUPS, body, 0)
    plsc.subcore_barrier()
    pltpu.sync_copy(acc.at[pl.ds(s * rpt, rpt)],
                    out_hbm.at[c, pl.ds(s * rpt, rpt)])


def _sigmoid(v):
    return 1.0 / (1.0 + jnp.exp(-v))


def _tc_prep_body(x_ref, win_ref, bin_ref, a1_ref, a2_ref, b1w_ref, b2w_ref,
                  bc1_ref, p_ref, q_ref, xa_ref, xb_ref):
    x = x_ref[...]
    xa = jnp.dot(x, a2_ref[...], preferred_element_type=jnp.float32)
    xb = jnp.dot(x, b2w_ref[...], preferred_element_type=jnp.float32)
    h0 = jnp.tanh(jnp.dot(x, win_ref[...], preferred_element_type=jnp.float32)
                  + bin_ref[...])
    xa_ref[...] = xa
    xb_ref[...] = xb
    p_ref[...] = (jnp.dot(h0, a1_ref[...], preferred_element_type=jnp.float32)
                  + xa + bc1_ref[...])
    q_ref[...] = (jnp.dot(h0, b1w_ref[...], preferred_element_type=jnp.float32)
                  + xb)


def _tc_mid_body(pg_ref, qg_ref, w2_ref, b2_ref, u_ref):
    t = _sigmoid(pg_ref[...] + qg_ref[...])
    u_ref[...] = _sigmoid(
        jnp.dot(t, w2_ref[...], preferred_element_type=jnp.float32)
        + b2_ref[...])


def _tc_pq_body(hp_ref, xa_ref, xb_ref, a1_ref, b1w_ref, bc1_ref,
                p_ref, q_ref):
    hn = hp_ref[0] + hp_ref[1]
    p_ref[...] = (jnp.dot(hn, a1_ref[...], preferred_element_type=jnp.float32)
                  + xa_ref[...] + bc1_ref[...])
    q_ref[...] = (jnp.dot(hn, b1w_ref[...], preferred_element_type=jnp.float32)
                  + xb_ref[...])


def _tc_final_body(hp_ref, x_ref, oh_ref, woh_ref, wox_ref, bo_ref,
                   out_ref, acch, accx):
    i = pl.program_id(0)

    @pl.when(i == 0)
    def _():
        acch[...] = jnp.zeros_like(acch)
        accx[...] = jnp.zeros_like(accx)

    hn = hp_ref[0] + hp_ref[1]
    oh = oh_ref[...]
    dn = (((0,), (0,)), ((), ()))
    acch[...] += lax.dot_general(oh, hn, dn,
                                 preferred_element_type=jnp.float32)
    accx[...] += lax.dot_general(oh, x_ref[...], dn,
                                 preferred_element_type=jnp.float32)

    @pl.when(i == (NP // BN) - 1)
    def _():
        out_ref[...] = (
            jnp.dot(acch[...], woh_ref[...],
                    preferred_element_type=jnp.float32)
            + jnp.dot(accx[...], wox_ref[...],
                      preferred_element_type=jnp.float32)
            + bo_ref[...])


def _full(shape):
    return pl.BlockSpec(shape, lambda i: tuple(0 for _ in shape))


def kernel(x, edge_index, batch, W_in, b_in, W_c1, b_c1, W_c2, b_c2,
           W_out, b_out):
    f32 = jnp.float32
    xp = jnp.zeros((NP, IN_DIM), f32).at[:N].set(x)
    src = jnp.concatenate(
        [edge_index[0], jnp.zeros((EP - E,), jnp.int32)]).reshape(NCHUNK, G)
    dst = jnp.concatenate(
        [edge_index[1], jnp.full((EP - E,), N, jnp.int32)]).reshape(NCHUNK, G)
    A1 = W_c1[:HID] - W_c1[192:256]
    A2 = W_c1[HID:192] - W_c1[256:]
    B1 = W_c1[192:256]
    B2 = W_c1[256:]
    bc1 = b_c1.reshape(1, HID)
    b2 = b_c2.reshape(1, HID)
    bo = b_out.reshape(1, OUT_DIM)
    woh = W_out[:HID]
    wox = W_out[HID:]
    batch_p = jnp.concatenate(
        [batch, jnp.full((NP - N,), N_GRAPHS, jnp.int32)])
    oh = (batch_p[:, None] == jnp.arange(N_GRAPHS)[None, :]).astype(f32)
    counts = jnp.maximum(oh.sum(axis=0), 1.0)
    ohs = oh / counts[None, :]
    zz = jnp.zeros((NP, HID), f32)

    nblk = NP // BN
    P, Q, XA, XB = pl.pallas_call(
        _tc_prep_body,
        grid=(nblk,),
        in_specs=[
            pl.BlockSpec((BN, IN_DIM), lambda i: (i, 0)),
            _full((IN_DIM, HID)),
            _full((1, HID)),
            _full((HID, HID)),
            _full((IN_DIM, HID)),
            _full((HID, HID)),
            _full((IN_DIM, HID)),
            _full((1, HID)),
        ],
        out_specs=[pl.BlockSpec((BN, HID), lambda i: (i, 0))] * 4,
        out_shape=[jax.ShapeDtypeStruct((NP, HID), f32)] * 4,
    )(xp, W_in, b_in.reshape(1, HID), A1, A2, B1, B2, bc1)

    mid = pl.pallas_call(
        _tc_mid_body,
        grid=(EP // BE,),
        in_specs=[
            pl.BlockSpec((BE, HID), lambda i: (i, 0)),
            pl.BlockSpec((BE, HID), lambda i: (i, 0)),
            _full((HID, HID)),
            _full((1, HID)),
        ],
        out_specs=pl.BlockSpec((BE, HID), lambda i: (i, 0)),
        out_shape=jax.ShapeDtypeStruct((EP, HID), f32),
    )

    pq = pl.pallas_call(
        _tc_pq_body,
        grid=(nblk,),
        in_specs=[
            pl.BlockSpec((2, BN, HID), lambda i: (0, i, 0)),
            pl.BlockSpec((BN, HID), lambda i: (i, 0)),
            pl.BlockSpec((BN, HID), lambda i: (i, 0)),
            _full((HID, HID)),
            _full((HID, HID)),
            _full((1, HID)),
        ],
        out_specs=[pl.BlockSpec((BN, HID), lambda i: (i, 0))] * 2,
        out_shape=[jax.ShapeDtypeStruct((NP, HID), f32)] * 2,
    )

    hp = None
    for it in range(N_ITERS):
        pg, qg = _sc_gather(P, Q, dst, src)
        u = mid(pg, qg, W_c2, b2)
        hp = _sc_scatter(u, dst, zz)
        if it < N_ITERS - 1:
            P, Q = pq(hp, XA, XB, A1, B1, bc1)

    out = pl.pallas_call(
        _tc_final_body,
        grid=(nblk,),
        in_specs=[
            pl.BlockSpec((2, BN, HID), lambda i: (0, i, 0)),
            pl.BlockSpec((BN, IN_DIM), lambda i: (i, 0)),
            pl.BlockSpec((BN, N_GRAPHS), lambda i: (i, 0)),
            _full((HID, OUT_DIM)),
            _full((IN_DIM, OUT_DIM)),
            _full((1, OUT_DIM)),
        ],
        out_specs=_full((N_GRAPHS, OUT_DIM)),
        out_shape=jax.ShapeDtypeStruct((N_GRAPHS, OUT_DIM), f32),
        scratch_shapes=[
            pltpu.VMEM((N_GRAPHS, HID), f32),
            pltpu.VMEM((N_GRAPHS, IN_DIM), f32),
        ],
    )(hp, xp, ohs, woh, wox, bo)
    return out


# speedup vs baseline: 2.6115x; 1.1071x over previous
import functools

import jax
import jax.numpy as jnp
from jax import lax
from jax.experimental import pallas as pl
from jax.experimental.pallas import tpu as pltpu
from jax.experimental.pallas import tpu_sc as plsc

N = 10000
NP = 10240
E = 320000
EP = 327680
G = 128
NCHUNK = EP // G
NWORK = 32
JG = 8
GG = NCHUNK // (NWORK * JG)
JS = 5
GS = NCHUNK // (NWORK * JS)
IN_DIM = 128
HID = 64
N_ITERS = 10
N_GRAPHS = 16
OUT_DIM = 16

BE = 2048
BN = 1024

_mesh = plsc.VectorSubcoreMesh(core_axis_name="c", subcore_axis_name="s")
_sc_params = pltpu.CompilerParams(use_tc_tiling_on_sc=False)
bf16 = jnp.bfloat16


@functools.partial(
    pl.kernel,
    out_type=[
        jax.ShapeDtypeStruct((EP, HID), bf16),
        jax.ShapeDtypeStruct((EP, HID), bf16),
    ],
    mesh=_mesh,
    compiler_params=_sc_params,
    scratch_types=[
        pltpu.VMEM((2, JG, G), jnp.int32),
        pltpu.VMEM((2, JG, G), jnp.int32),
        pltpu.VMEM((JG * G, HID), bf16),
        pltpu.VMEM((JG * G, HID), bf16),
        pltpu.SemaphoreType.DMA,
        pltpu.SemaphoreType.DMA,
    ],
)
def _sc_gather(p_hbm, q_hbm, dst_hbm, src_hbm, pg_hbm, qg_hbm,
               dv, sv, pr, qr, sem_i, sem_g):
    c = lax.axis_index("c")
    s = lax.axis_index("s")
    wid = s * 2 + c
    row00 = wid * GG * JG

    def idx_fetch(g, buf):
        row0 = row00 + g * JG
        pltpu.async_copy(dst_hbm.at[pl.ds(row0, JG)], dv.at[buf], sem_i)
        pltpu.async_copy(src_hbm.at[pl.ds(row0, JG)], sv.at[buf], sem_i)

    def idx_wait(buf):
        pltpu.make_async_copy(
            dst_hbm.at[pl.ds(0, JG)], dv.at[buf], sem_i).wait()
        pltpu.make_async_copy(
            src_hbm.at[pl.ds(0, JG)], sv.at[buf], sem_i).wait()

    idx_fetch(0, 0)

    def body(g, carry):
        cur = lax.rem(g, 2)
        nxt = 1 - cur
        idx_wait(cur)
        idx_fetch(jnp.minimum(g + 1, GG - 1), nxt)
        for j in range(JG):
            pltpu.async_copy(p_hbm.at[dv.at[cur, j]],
                             pr.at[pl.ds(j * G, G)], sem_g)
            pltpu.async_copy(q_hbm.at[sv.at[cur, j]],
                             qr.at[pl.ds(j * G, G)], sem_g)
        for j in range(JG):
            pltpu.make_async_copy(p_hbm.at[dv.at[cur, j]],
                                  pr.at[pl.ds(j * G, G)], sem_g).wait()
            pltpu.make_async_copy(q_hbm.at[sv.at[cur, j]],
                                  qr.at[pl.ds(j * G, G)], sem_g).wait()
        base = (row00 + g * JG) * G
        pltpu.sync_copy(pr, pg_hbm.at[pl.ds(base, JG * G)])
        pltpu.sync_copy(qr, qg_hbm.at[pl.ds(base, JG * G)])
        return carry

    lax.fori_loop(0, GG, body, 0)
    idx_wait(GG % 2)


@functools.partial(
    pl.kernel,
    out_type=jax.ShapeDtypeStruct((2, NP, HID), jnp.float32),
    mesh=_mesh,
    compiler_params=_sc_params,
    scratch_types=[
        pltpu.VMEM((2, JS, G), jnp.int32),
        pltpu.VMEM((2, JS * G, HID), jnp.float32),
        pltpu.VMEM_SHARED((NP, HID), jnp.float32),
        pltpu.SemaphoreType.DMA,
        pltpu.SemaphoreType.DMA,
    ],
)
def _sc_scatter(u_hbm, dst_hbm, zeros_hbm, out_hbm, dv, uv, acc,
                sem_i, sem_s):
    c = lax.axis_index("c")
    s = lax.axis_index("s")
    rpt = NP // 16
    row00 = (c * 16 + s) * GS * JS

    def fetch(g, buf):
        row0 = row00 + g * JS
        pltpu.async_copy(dst_hbm.at[pl.ds(row0, JS)], dv.at[buf], sem_i)
        pltpu.async_copy(u_hbm.at[pl.ds(row0 * G, JS * G)], uv.at[buf],
                         sem_i)

    def fetch_wait(buf):
        pltpu.make_async_copy(
            dst_hbm.at[pl.ds(0, JS)], dv.at[buf], sem_i).wait()
        pltpu.make_async_copy(
            u_hbm.at[pl.ds(0, JS * G)], uv.at[buf], sem_i).wait()

    pltpu.sync_copy(zeros_hbm.at[pl.ds(s * rpt, rpt)],
                    acc.at[pl.ds(s * rpt, rpt)])
    fetch(0, 0)
    plsc.subcore_barrier()

    def body(g, carry):
        cur = lax.rem(g, 2)
        nxt = 1 - cur
        fetch_wait(cur)
        for j in range(JS):
            pltpu.async_copy(uv.at[cur, pl.ds(j * G, G)],
                             acc.at[dv.at[cur, j]], sem_s, add=True)
        fetch(jnp.minimum(g + 1, GS - 1), nxt)
        for j in range(JS):
            pltpu.make_async_copy(uv.at[cur, pl.ds(j * G, G)],
                                  acc.at[dv.at[cur, j]], sem_s).wait()
        return carry

    lax.fori_loop(0, GS, body, 0)
    fetch_wait(GS % 2)
    plsc.subcore_barrier()
    pltpu.sync_copy(acc.at[pl.ds(s * rpt, rpt)],
                    out_hbm.at[c, pl.ds(s * rpt, rpt)])


def _sigmoid(v):
    return 1.0 / (1.0 + jnp.exp(-v))


def _tc_prep_body(x_ref, win_ref, bin_ref, a1_ref, a2_ref, b1w_ref, b2w_ref,
                  bc1_ref, p_ref, q_ref, xa_ref, xb_ref):
    x = x_ref[...]
    xa = jnp.dot(x, a2_ref[...], preferred_element_type=jnp.float32)
    xb = jnp.dot(x, b2w_ref[...], preferred_element_type=jnp.float32)
    h0 = jnp.tanh(jnp.dot(x, win_ref[...], preferred_element_type=jnp.float32)
                  + bin_ref[...])
    xa_ref[...] = xa
    xb_ref[...] = xb
    p_ref[...] = (jnp.dot(h0, a1_ref[...], preferred_element_type=jnp.float32)
                  + xa + bc1_ref[...]).astype(bf16)
    q_ref[...] = (jnp.dot(h0, b1w_ref[...], preferred_element_type=jnp.float32)
                  + xb).astype(bf16)


def _tc_mid_body(pg_ref, qg_ref, w2_ref, b2_ref, u_ref):
    t = _sigmoid(pg_ref[...].astype(jnp.float32)
                 + qg_ref[...].astype(jnp.float32))
    u_ref[...] = _sigmoid(
        jnp.dot(t.astype(bf16), w2_ref[...],
                preferred_element_type=jnp.float32)
        + b2_ref[...])


def _tc_pq_body(hp_ref, xa_ref, xb_ref, a1_ref, b1w_ref, bc1_ref,
                p_ref, q_ref):
    hn = hp_ref[0] + hp_ref[1]
    p_ref[...] = (jnp.dot(hn, a1_ref[...], preferred_element_type=jnp.float32)
                  + xa_ref[...] + bc1_ref[...]).astype(bf16)
    q_ref[...] = (jnp.dot(hn, b1w_ref[...], preferred_element_type=jnp.float32)
                  + xb_ref[...]).astype(bf16)


def _tc_final_body(hp_ref, x_ref, oh_ref, woh_ref, wox_ref, bo_ref,
                   out_ref, acch, accx):
    i = pl.program_id(0)

    @pl.when(i == 0)
    def _():
        acch[...] = jnp.zeros_like(acch)
        accx[...] = jnp.zeros_like(accx)

    hn = hp_ref[0] + hp_ref[1]
    oh = oh_ref[...]
    dn = (((0,), (0,)), ((), ()))
    acch[...] += lax.dot_general(oh, hn, dn,
                                 preferred_element_type=jnp.float32)
    accx[...] += lax.dot_general(oh, x_ref[...], dn,
                                 preferred_element_type=jnp.float32)

    @pl.when(i == (NP // BN) - 1)
    def _():
        out_ref[...] = (
            jnp.dot(acch[...], woh_ref[...],
                    preferred_element_type=jnp.float32)
            + jnp.dot(accx[...], wox_ref[...],
                      preferred_element_type=jnp.float32)
            + bo_ref[...])


def _full(shape):
    return pl.BlockSpec(shape, lambda i: tuple(0 for _ in shape))


def kernel(x, edge_index, batch, W_in, b_in, W_c1, b_c1, W_c2, b_c2,
           W_out, b_out):
    f32 = jnp.float32
    xp = jnp.zeros((NP, IN_DIM), f32).at[:N].set(x)
    src = jnp.concatenate(
        [edge_index[0], jnp.zeros((EP - E,), jnp.int32)]).reshape(NCHUNK, G)
    dst = jnp.concatenate(
        [edge_index[1], jnp.full((EP - E,), N, jnp.int32)]).reshape(NCHUNK, G)
    A1 = W_c1[:HID] - W_c1[192:256]
    A2 = W_c1[HID:192] - W_c1[256:]
    B1 = W_c1[192:256]
    B2 = W_c1[256:]
    bc1 = b_c1.reshape(1, HID)
    b2 = b_c2.reshape(1, HID)
    bo = b_out.reshape(1, OUT_DIM)
    woh = W_out[:HID]
    wox = W_out[HID:]
    w2b = W_c2.astype(bf16)
    batch_p = jnp.concatenate(
        [batch, jnp.full((NP - N,), N_GRAPHS, jnp.int32)])
    oh = (batch_p[:, None] == jnp.arange(N_GRAPHS)[None, :]).astype(f32)
    counts = jnp.maximum(oh.sum(axis=0), 1.0)
    ohs = oh / counts[None, :]
    zz = jnp.zeros((NP, HID), f32)

    nblk = NP // BN
    P, Q, XA, XB = pl.pallas_call(
        _tc_prep_body,
        grid=(nblk,),
        in_specs=[
            pl.BlockSpec((BN, IN_DIM), lambda i: (i, 0)),
            _full((IN_DIM, HID)),
            _full((1, HID)),
            _full((HID, HID)),
            _full((IN_DIM, HID)),
            _full((HID, HID)),
            _full((IN_DIM, HID)),
            _full((1, HID)),
        ],
        out_specs=[pl.BlockSpec((BN, HID), lambda i: (i, 0))] * 4,
        out_shape=[jax.ShapeDtypeStruct((NP, HID), bf16)] * 2
        + [jax.ShapeDtypeStruct((NP, HID), f32)] * 2,
    )(xp, W_in, b_in.reshape(1, HID), A1, A2, B1, B2, bc1)

    mid = pl.pallas_call(
        _tc_mid_body,
        grid=(EP // BE,),
        in_specs=[
            pl.BlockSpec((BE, HID), lambda i: (i, 0)),
            pl.BlockSpec((BE, HID), lambda i: (i, 0)),
            _full((HID, HID)),
            _full((1, HID)),
        ],
        out_specs=pl.BlockSpec((BE, HID), lambda i: (i, 0)),
        out_shape=jax.ShapeDtypeStruct((EP, HID), f32),
    )

    pq = pl.pallas_call(
        _tc_pq_body,
        grid=(nblk,),
        in_specs=[
            pl.BlockSpec((2, BN, HID), lambda i: (0, i, 0)),
            pl.BlockSpec((BN, HID), lambda i: (i, 0)),
            pl.BlockSpec((BN, HID), lambda i: (i, 0)),
            _full((HID, HID)),
            _full((HID, HID)),
            _full((1, HID)),
        ],
        out_specs=[pl.BlockSpec((BN, HID), lambda i: (i, 0))] * 2,
        out_shape=[jax.ShapeDtypeStruct((NP, HID), bf16)] * 2,
    )

    hp = None
    for it in range(N_ITERS):
        pg, qg = _sc_gather(P, Q, dst, src)
        u = mid(pg, qg, w2b, b2)
        hp = _sc_scatter(u, dst, zz)
        if it < N_ITERS - 1:
            P, Q = pq(hp, XA, XB, A1, B1, bc1)

    out = pl.pallas_call(
        _tc_final_body,
        grid=(nblk,),
        in_specs=[
            pl.BlockSpec((2, BN, HID), lambda i: (0, i, 0)),
            pl.BlockSpec((BN, IN_DIM), lambda i: (i, 0)),
            pl.BlockSpec((BN, N_GRAPHS), lambda i: (i, 0)),
            _full((HID, OUT_DIM)),
            _full((IN_DIM, OUT_DIM)),
            _full((1, OUT_DIM)),
        ],
        out_specs=_full((N_GRAPHS, OUT_DIM)),
        out_shape=jax.ShapeDtypeStruct((N_GRAPHS, OUT_DIM), f32),
        scratch_shapes=[
            pltpu.VMEM((N_GRAPHS, HID), f32),
            pltpu.VMEM((N_GRAPHS, IN_DIM), f32),
        ],
    )(hp, xp, ohs, woh, wox, bo)
    return out
